# Initial kernel scaffold; baseline (speedup 1.0000x reference)
import jax
import jax.numpy as jnp
from jax.experimental import pallas as pl


def kernel(model_id, prompt_id, edge_index, P, Q, W_proj, b_proj, Wl1, Wr1, b1, Wl2, Wr2, b2, W_text, b_text, W_cls, b_cls):
    raise NotImplementedError("write your pallas kernel here")


# speedup vs baseline: 1.0560x; 1.0560x over previous
import functools

import jax
import jax.numpy as jnp
from jax import lax
from jax.experimental import pallas as pl
from jax.experimental.pallas import tpu as pltpu


def _sage(x, src, dst, Wl, Wr, b):
    n = x.shape[0]
    msgs = jnp.take(x, src, axis=0)
    sums = jnp.zeros((n, x.shape[1]), x.dtype).at[dst].add(msgs)
    cnt = jnp.zeros((n,), x.dtype).at[dst].add(1.0)
    aggr = sums / jnp.clip(cnt, 1.0)[:, None]
    return aggr @ Wl + x @ Wr + b


def _tc4_body(q_ref, mr_ref, wt_ref, bt_ref, wc_ref, bc_ref, out_ref):
    pr = jnp.dot(q_ref[...], wt_ref[...], preferred_element_type=jnp.float32) + bt_ref[...]
    t = mr_ref[...] * pr
    out_ref[...] = jnp.dot(t, wc_ref[...], preferred_element_type=jnp.float32) + bc_ref[...]


def kernel(model_id, prompt_id, edge_index, P, Q, W_proj, b_proj,
           Wl1, Wr1, b1, Wl2, Wr2, b2, W_text, b_text, W_cls, b_cls):
    model_id = model_id.astype(jnp.int32)
    prompt_id = prompt_id.astype(jnp.int32)
    src = edge_index[0].astype(jnp.int32)
    dst = edge_index[1].astype(jnp.int32)

    x = P @ W_proj + b_proj
    h = jax.nn.relu(_sage(x, src, dst, Wl1, Wr1, b1))
    h = _sage(h, src, dst, Wl2, Wr2, b2)
    model_repr = jnp.take(h, model_id, axis=0)
    qg = jnp.take(Q, prompt_id, axis=0)

    B = qg.shape[0]
    D = 232
    DP = 256
    mrP = jnp.pad(model_repr, ((0, 0), (0, DP - D)))
    wtP = jnp.pad(W_text, ((0, 0), (0, DP - D)))
    btP = jnp.pad(b_text, (0, DP - D)).reshape(1, DP)
    wcP = jnp.pad(W_cls, ((0, DP - D), (0, 128 - W_cls.shape[1])))
    bcP = jnp.pad(b_cls, (0, 128 - b_cls.shape[0])).reshape(1, 128)

    RB = 512
    out = pl.pallas_call(
        _tc4_body,
        grid=(B // RB,),
        in_specs=[
            pl.BlockSpec((RB, 768), lambda i: (i, 0)),
            pl.BlockSpec((RB, DP), lambda i: (i, 0)),
            pl.BlockSpec((768, DP), lambda i: (0, 0)),
            pl.BlockSpec((1, DP), lambda i: (0, 0)),
            pl.BlockSpec((DP, 128), lambda i: (0, 0)),
            pl.BlockSpec((1, 128), lambda i: (0, 0)),
        ],
        out_specs=pl.BlockSpec((RB, 128), lambda i: (i, 0)),
        out_shape=jax.ShapeDtypeStruct((B, 128), jnp.float32),
    )(qg, mrP, wtP, btP, wcP, bcP)
    return out[:, :2]


# speedup vs baseline: 2.9903x; 2.8318x over previous
import functools

import jax
import jax.numpy as jnp
from jax import lax
from jax.experimental import pallas as pl
from jax.experimental.pallas import tpu as pltpu
from jax.experimental.pallas import tpu_sc as plsc

N = 10000
NPAD = 10240
E = 160000
EPAD = 163840
ECH = EPAD // 128
B = 16384
TEXT = 768

_MESH = plsc.VectorSubcoreMesh(core_axis_name="c", subcore_axis_name="s")


def _proj_body(p_ref, w_ref, b_ref, lo_ref, hi_ref):
    acc = jnp.dot(p_ref[...], w_ref[...], preferred_element_type=jnp.float32)
    acc = acc + b_ref[...]
    lo_ref[...] = acc[:, :128]
    hi_ref[...] = acc[:, 128:]


def _tc_proj(P, WprojP, bprojP):
    R = 1024
    return pl.pallas_call(
        _proj_body,
        grid=(NPAD // R,),
        in_specs=[
            pl.BlockSpec((R, 256), lambda i: (i, 0)),
            pl.BlockSpec((256, 256), lambda i: (0, 0)),
            pl.BlockSpec((1, 256), lambda i: (0, 0)),
        ],
        out_specs=[
            pl.BlockSpec((R, 128), lambda i: (i, 0)),
            pl.BlockSpec((R, 128), lambda i: (i, 0)),
        ],
        out_shape=[
            jax.ShapeDtypeStruct((NPAD, 128), jnp.float32),
            jax.ShapeDtypeStruct((NPAD, 128), jnp.float32),
        ],
    )(P, WprojP, bprojP)


def _conv_body(relu, slo_ref, shi_ref, xlo_ref, xhi_ref, cnt_ref,
               wla_ref, wlb_ref, wra_ref, wrb_ref, b_ref, lo_ref, hi_ref):
    c = jnp.maximum(cnt_ref[...], 1.0)
    alo = slo_ref[...] / c
    ahi = shi_ref[...] / c
    h = (jnp.dot(alo, wla_ref[...], preferred_element_type=jnp.float32)
         + jnp.dot(ahi, wlb_ref[...], preferred_element_type=jnp.float32)
         + jnp.dot(xlo_ref[...], wra_ref[...], preferred_element_type=jnp.float32)
         + jnp.dot(xhi_ref[...], wrb_ref[...], preferred_element_type=jnp.float32)
         + b_ref[...])
    if relu:
        h = jnp.maximum(h, 0.0)
    lo_ref[...] = h[:, :128]
    hi_ref[...] = h[:, 128:]


def _tc_conv(slo, shi, xlo, xhi, cntc, WlP, WrP, br, relu):
    R = 1024
    return pl.pallas_call(
        functools.partial(_conv_body, relu),
        grid=(NPAD // R,),
        in_specs=[
            pl.BlockSpec((R, 128), lambda i: (i, 0)),
            pl.BlockSpec((R, 128), lambda i: (i, 0)),
            pl.BlockSpec((R, 128), lambda i: (i, 0)),
            pl.BlockSpec((R, 128), lambda i: (i, 0)),
            pl.BlockSpec((R, 1), lambda i: (i, 0)),
            pl.BlockSpec((128, 256), lambda i: (0, 0)),
            pl.BlockSpec((128, 256), lambda i: (0, 0)),
            pl.BlockSpec((128, 256), lambda i: (0, 0)),
            pl.BlockSpec((128, 256), lambda i: (0, 0)),
            pl.BlockSpec((1, 256), lambda i: (0, 0)),
        ],
        out_specs=[
            pl.BlockSpec((R, 128), lambda i: (i, 0)),
            pl.BlockSpec((R, 128), lambda i: (i, 0)),
        ],
        out_shape=[
            jax.ShapeDtypeStruct((NPAD, 128), jnp.float32),
            jax.ShapeDtypeStruct((NPAD, 128), jnp.float32),
        ],
    )(slo, shi, xlo, xhi, cntc, WlP[:128], WlP[128:], WrP[:128], WrP[128:], br)


def _final_body(q_ref, mlo_ref, mhi_ref, wt_ref, bt_ref, wc_ref, bc_ref, out_ref):
    pr = jnp.dot(q_ref[...], wt_ref[...], preferred_element_type=jnp.float32)
    pr = pr + bt_ref[...]
    tlo = mlo_ref[...] * pr[:, :128]
    thi = mhi_ref[...] * pr[:, 128:]
    out_ref[...] = (jnp.dot(tlo, wc_ref[:128], preferred_element_type=jnp.float32)
                    + jnp.dot(thi, wc_ref[128:], preferred_element_type=jnp.float32)
                    + bc_ref[...])


def _tc_final(qg, mlo, mhi, WtP, btP, WcP, bcP):
    R = 512
    return pl.pallas_call(
        _final_body,
        grid=(B // R,),
        in_specs=[
            pl.BlockSpec((R, TEXT), lambda i: (i, 0)),
            pl.BlockSpec((R, 128), lambda i: (i, 0)),
            pl.BlockSpec((R, 128), lambda i: (i, 0)),
            pl.BlockSpec((TEXT, 256), lambda i: (0, 0)),
            pl.BlockSpec((1, 256), lambda i: (0, 0)),
            pl.BlockSpec((256, 128), lambda i: (0, 0)),
            pl.BlockSpec((1, 128), lambda i: (0, 0)),
        ],
        out_specs=pl.BlockSpec((R, 128), lambda i: (i, 0)),
        out_shape=jax.ShapeDtypeStruct((B, 128), jnp.float32),
    )(qg, mlo, mhi, WtP, btP, WcP, bcP)


def _segsum_body(tlo, thi, src2, dst2, zf, z1, ones,
                 out_lo, out_hi, cnt_out,
                 src_v, dst_v, rows_v, z_v, z1_v, ones_v, acc, cnt_acc, sem):
    cid = lax.axis_index("c")
    sid = lax.axis_index("s")

    pltpu.sync_copy(src2.at[pl.ds(sid * 80, 80)], src_v)
    pltpu.sync_copy(dst2.at[pl.ds(sid * 80, 80)], dst_v)
    pltpu.sync_copy(zf, z_v)
    for j in range(10):
        pltpu.sync_copy(z_v, acc.at[pl.ds(sid * 640 + j * 64, 64)])
    pltpu.sync_copy(z1, z1_v)
    pltpu.sync_copy(z1_v, cnt_acc.at[pl.ds(sid * 640, 640)])
    pltpu.sync_copy(ones, ones_v)
    plsc.subcore_barrier()

    def run(table):
        def step(j, carry):
            pltpu.async_copy(table.at[src_v.at[j]], rows_v, sem).wait()
            pltpu.sync_copy(rows_v, acc.at[dst_v.at[j]], add=True)
            pltpu.sync_copy(ones_v, cnt_acc.at[dst_v.at[j]], add=True)
            return carry
        lax.fori_loop(0, 80, step, 0)

    @pl.when(cid == 0)
    def _():
        run(tlo)

    @pl.when(cid == 1)
    def _():
        run(thi)

    plsc.subcore_barrier()

    @pl.when(cid == 0)
    def _():
        pltpu.sync_copy(acc.at[pl.ds(sid * 640, 640)],
                        out_lo.at[pl.ds(sid * 640, 640)])
        pltpu.sync_copy(cnt_acc.at[pl.ds(sid * 640, 640)],
                        cnt_out.at[pl.ds(sid * 640, 640)])

    @pl.when(cid == 1)
    def _():
        pltpu.sync_copy(acc.at[pl.ds(sid * 640, 640)],
                        out_hi.at[pl.ds(sid * 640, 640)])


_sc_segsum = pl.kernel(
    _segsum_body,
    out_type=[
        jax.ShapeDtypeStruct((NPAD, 128), jnp.float32),
        jax.ShapeDtypeStruct((NPAD, 128), jnp.float32),
        jax.ShapeDtypeStruct((NPAD,), jnp.float32),
    ],
    mesh=_MESH,
    scratch_types=[
        pltpu.VMEM((80, 128), jnp.int32),
        pltpu.VMEM((80, 128), jnp.int32),
        pltpu.VMEM((128, 128), jnp.float32),
        pltpu.VMEM((64, 128), jnp.float32),
        pltpu.VMEM((640,), jnp.float32),
        pltpu.VMEM((128,), jnp.float32),
        pltpu.VMEM_SHARED((NPAD, 128), jnp.float32),
        pltpu.VMEM_SHARED((NPAD,), jnp.float32),
        pltpu.SemaphoreType.DMA,
    ],
)


def _gather_model_body(tlo, thi, idx2, out_lo, out_hi, idx_v, rows_v, sem):
    cid = lax.axis_index("c")
    sid = lax.axis_index("s")
    pltpu.sync_copy(idx2.at[pl.ds(sid * 8, 8)], idx_v)

    def run(table, out):
        def step(j, carry):
            pltpu.async_copy(table.at[idx_v.at[j]], rows_v, sem).wait()
            pltpu.sync_copy(rows_v, out.at[pl.ds(sid * 1024 + j * 128, 128)])
            return carry
        lax.fori_loop(0, 8, step, 0)

    @pl.when(cid == 0)
    def _():
        run(tlo, out_lo)

    @pl.when(cid == 1)
    def _():
        run(thi, out_hi)


_sc_gather_model = pl.kernel(
    _gather_model_body,
    out_type=[
        jax.ShapeDtypeStruct((B, 128), jnp.float32),
        jax.ShapeDtypeStruct((B, 128), jnp.float32),
    ],
    mesh=_MESH,
    scratch_types=[
        pltpu.VMEM((8, 128), jnp.int32),
        pltpu.VMEM((128, 128), jnp.float32),
        pltpu.SemaphoreType.DMA,
    ],
)


def _gather_q_body(q_hbm, idx2, out, idx_v, rows_v, sem):
    cid = lax.axis_index("c")
    sid = lax.axis_index("s")
    wid = sid * 2 + cid
    pltpu.sync_copy(idx2.at[pl.ds(wid * 8, 8)], idx_v)

    def step(j, carry):
        pltpu.async_copy(q_hbm.at[idx_v.at[j]], rows_v, sem).wait()
        pltpu.sync_copy(rows_v, out.at[pl.ds(wid * 512 + j * 64, 64)])
        return carry
    lax.fori_loop(0, 8, step, 0)


_sc_gather_q = pl.kernel(
    _gather_q_body,
    out_type=jax.ShapeDtypeStruct((B, TEXT), jnp.float32),
    mesh=_MESH,
    scratch_types=[
        pltpu.VMEM((8, 64), jnp.int32),
        pltpu.VMEM((64, TEXT), jnp.float32),
        pltpu.SemaphoreType.DMA,
    ],
)


def kernel(model_id, prompt_id, edge_index, P, Q, W_proj, b_proj,
           Wl1, Wr1, b1, Wl2, Wr2, b2, W_text, b_text, W_cls, b_cls):
    f32 = jnp.float32
    model_id = model_id.astype(jnp.int32)
    prompt_id = prompt_id.astype(jnp.int32)
    src = edge_index[0].astype(jnp.int32)
    dst = edge_index[1].astype(jnp.int32)
    src2d = jnp.concatenate([src, jnp.zeros((EPAD - E,), jnp.int32)]).reshape(ECH, 128)
    dst2d = jnp.concatenate([dst, jnp.full((EPAD - E,), N, jnp.int32)]).reshape(ECH, 128)
    mid2d = model_id.reshape(128, 128)
    pid2d = prompt_id.reshape(256, 64)

    WprojP = jnp.pad(W_proj, ((0, 0), (0, 24)))
    bprojP = jnp.pad(b_proj, (0, 24)).reshape(1, 256)
    Wl1P = jnp.pad(Wl1, ((0, 24), (0, 0)))
    Wr1P = jnp.pad(Wr1, ((0, 24), (0, 0)))
    b1r = b1.reshape(1, 256)
    Wl2P = jnp.pad(Wl2, ((0, 0), (0, 24)))
    Wr2P = jnp.pad(Wr2, ((0, 0), (0, 24)))
    b2P = jnp.pad(b2, (0, 24)).reshape(1, 256)
    WtP = jnp.pad(W_text, ((0, 0), (0, 24)))
    btP = jnp.pad(b_text, (0, 24)).reshape(1, 256)
    WcP = jnp.pad(W_cls, ((0, 24), (0, 128 - W_cls.shape[1])))
    bcP = jnp.pad(b_cls, (0, 128 - b_cls.shape[0])).reshape(1, 128)

    zf = jnp.zeros((64, 128), f32)
    z1 = jnp.zeros((640,), f32)
    ones = jnp.ones((128,), f32)

    x_lo, x_hi = _tc_proj(P, WprojP, bprojP)
    s1_lo, s1_hi, cnt = _sc_segsum(x_lo, x_hi, src2d, dst2d, zf, z1, ones)
    cntc = cnt.reshape(NPAD, 1)
    h1_lo, h1_hi = _tc_conv(s1_lo, s1_hi, x_lo, x_hi, cntc, Wl1P, Wr1P, b1r, True)
    s2_lo, s2_hi, _ = _sc_segsum(h1_lo, h1_hi, src2d, dst2d, zf, z1, ones)
    h2_lo, h2_hi = _tc_conv(s2_lo, s2_hi, h1_lo, h1_hi, cntc, Wl2P, Wr2P, b2P, False)
    mr_lo, mr_hi = _sc_gather_model(h2_lo, h2_hi, mid2d)
    qg = _sc_gather_q(Q, pid2d)
    out = _tc_final(qg, mr_lo, mr_hi, WtP, btP, WcP, bcP)
    return out[:, :2]


# speedup vs baseline: 3.2578x; 1.0895x over previous
import functools

import jax
import jax.numpy as jnp
from jax import lax
from jax.experimental import pallas as pl
from jax.experimental.pallas import tpu as pltpu
from jax.experimental.pallas import tpu_sc as plsc

N = 10000
NPAD = 10240
E = 160000
EPAD = 163840
CW = 128
CHN = 80
ECH = EPAD // CW
B = 16384
TEXT = 768

_MESH = plsc.VectorSubcoreMesh(core_axis_name="c", subcore_axis_name="s")


def _proj_body(p_ref, w_ref, b_ref, lo_ref, hi_ref):
    acc = jnp.dot(p_ref[...], w_ref[...], preferred_element_type=jnp.float32)
    acc = acc + b_ref[...]
    lo_ref[...] = acc[:, :128]
    hi_ref[...] = acc[:, 128:]


def _tc_proj(P, WprojP, bprojP):
    R = 1024
    return pl.pallas_call(
        _proj_body,
        grid=(NPAD // R,),
        in_specs=[
            pl.BlockSpec((R, 256), lambda i: (i, 0)),
            pl.BlockSpec((256, 256), lambda i: (0, 0)),
            pl.BlockSpec((1, 256), lambda i: (0, 0)),
        ],
        out_specs=[
            pl.BlockSpec((R, 128), lambda i: (i, 0)),
            pl.BlockSpec((R, 128), lambda i: (i, 0)),
        ],
        out_shape=[
            jax.ShapeDtypeStruct((NPAD, 128), jnp.float32),
            jax.ShapeDtypeStruct((NPAD, 128), jnp.float32),
        ],
    )(P, WprojP, bprojP)


def _conv_body(relu, slo_ref, shi_ref, xlo_ref, xhi_ref, cnt_ref,
               wla_ref, wlb_ref, wra_ref, wrb_ref, b_ref, lo_ref, hi_ref):
    c = jnp.maximum(cnt_ref[...], 1.0)
    alo = slo_ref[...] / c
    ahi = shi_ref[...] / c
    h = (jnp.dot(alo, wla_ref[...], preferred_element_type=jnp.float32)
         + jnp.dot(ahi, wlb_ref[...], preferred_element_type=jnp.float32)
         + jnp.dot(xlo_ref[...], wra_ref[...], preferred_element_type=jnp.float32)
         + jnp.dot(xhi_ref[...], wrb_ref[...], preferred_element_type=jnp.float32)
         + b_ref[...])
    if relu:
        h = jnp.maximum(h, 0.0)
    lo_ref[...] = h[:, :128]
    hi_ref[...] = h[:, 128:]


def _tc_conv(slo, shi, xlo, xhi, cntc, WlP, WrP, br, relu):
    R = 1024
    return pl.pallas_call(
        functools.partial(_conv_body, relu),
        grid=(NPAD // R,),
        in_specs=[
            pl.BlockSpec((R, 128), lambda i: (i, 0)),
            pl.BlockSpec((R, 128), lambda i: (i, 0)),
            pl.BlockSpec((R, 128), lambda i: (i, 0)),
            pl.BlockSpec((R, 128), lambda i: (i, 0)),
            pl.BlockSpec((R, 1), lambda i: (i, 0)),
            pl.BlockSpec((128, 256), lambda i: (0, 0)),
            pl.BlockSpec((128, 256), lambda i: (0, 0)),
            pl.BlockSpec((128, 256), lambda i: (0, 0)),
            pl.BlockSpec((128, 256), lambda i: (0, 0)),
            pl.BlockSpec((1, 256), lambda i: (0, 0)),
        ],
        out_specs=[
            pl.BlockSpec((R, 128), lambda i: (i, 0)),
            pl.BlockSpec((R, 128), lambda i: (i, 0)),
        ],
        out_shape=[
            jax.ShapeDtypeStruct((NPAD, 128), jnp.float32),
            jax.ShapeDtypeStruct((NPAD, 128), jnp.float32),
        ],
    )(slo, shi, xlo, xhi, cntc, WlP[:128], WlP[128:], WrP[:128], WrP[128:], br)


def _final_body(q_ref, mlo_ref, mhi_ref, wt_ref, bt_ref, wc_ref, bc_ref, out_ref):
    pr = jnp.dot(q_ref[...], wt_ref[...], preferred_element_type=jnp.float32)
    pr = pr + bt_ref[...]
    tlo = mlo_ref[...] * pr[:, :128]
    thi = mhi_ref[...] * pr[:, 128:]
    out_ref[...] = (jnp.dot(tlo, wc_ref[:128], preferred_element_type=jnp.float32)
                    + jnp.dot(thi, wc_ref[128:], preferred_element_type=jnp.float32)
                    + bc_ref[...])


def _tc_final(qg, mlo, mhi, WtP, btP, WcP, bcP):
    R = 512
    return pl.pallas_call(
        _final_body,
        grid=(B // R,),
        in_specs=[
            pl.BlockSpec((R, TEXT), lambda i: (i, 0)),
            pl.BlockSpec((R, 128), lambda i: (i, 0)),
            pl.BlockSpec((R, 128), lambda i: (i, 0)),
            pl.BlockSpec((TEXT, 256), lambda i: (0, 0)),
            pl.BlockSpec((1, 256), lambda i: (0, 0)),
            pl.BlockSpec((256, 128), lambda i: (0, 0)),
            pl.BlockSpec((1, 128), lambda i: (0, 0)),
        ],
        out_specs=pl.BlockSpec((R, 128), lambda i: (i, 0)),
        out_shape=jax.ShapeDtypeStruct((B, 128), jnp.float32),
    )(qg, mlo, mhi, WtP, btP, WcP, bcP)


def _segsum_body(with_counts, *refs):
    if with_counts:
        (tlo, thi, src2, dst2, zf, z1, ones,
         out_lo, out_hi, cnt_out,
         src_v, dst_v, rows0, rows1, z_v, z1_v, ones_v, acc, cnt_acc,
         sem0, sem1) = refs
    else:
        (tlo, thi, src2, dst2, zf,
         out_lo, out_hi,
         src_v, dst_v, rows0, rows1, z_v, acc,
         sem0, sem1) = refs
    cid = lax.axis_index("c")
    sid = lax.axis_index("s")

    pltpu.sync_copy(zf, z_v)
    for j in range(40):
        pltpu.sync_copy(z_v, acc.at[pl.ds(sid * 640 + j * 16, 16)])
    if with_counts:
        pltpu.sync_copy(z1, z1_v)
        pltpu.sync_copy(z1_v, cnt_acc.at[pl.ds(sid * 640, 640)])
        pltpu.sync_copy(ones, ones_v)
    plsc.subcore_barrier()

    def run(table):
        def half(h):
            pltpu.sync_copy(src2.at[pl.ds(sid * CHN + h * 40, 40)], src_v)
            pltpu.sync_copy(dst2.at[pl.ds(sid * CHN + h * 40, 40)], dst_v)
            pltpu.async_copy(table.at[src_v.at[0]], rows0, sem0)

            def step(t, carry):
                j0 = 2 * t
                pltpu.make_async_copy(table.at[src_v.at[j0]], rows0, sem0).wait()
                pltpu.async_copy(table.at[src_v.at[j0 + 1]], rows1, sem1)
                pltpu.sync_copy(rows0, acc.at[dst_v.at[j0]], add=True)
                if with_counts:
                    pltpu.sync_copy(ones_v, cnt_acc.at[dst_v.at[j0]], add=True)
                pltpu.make_async_copy(table.at[src_v.at[j0 + 1]], rows1, sem1).wait()

                @pl.when(j0 + 2 < 40)
                def _():
                    pltpu.async_copy(table.at[src_v.at[j0 + 2]], rows0, sem0)

                pltpu.sync_copy(rows1, acc.at[dst_v.at[j0 + 1]], add=True)
                if with_counts:
                    pltpu.sync_copy(ones_v, cnt_acc.at[dst_v.at[j0 + 1]], add=True)
                return carry
            lax.fori_loop(0, 20, step, 0)

        half(0)
        half(1)

    @pl.when(cid == 0)
    def _():
        run(tlo)

    @pl.when(cid == 1)
    def _():
        run(thi)

    plsc.subcore_barrier()

    @pl.when(cid == 0)
    def _():
        pltpu.sync_copy(acc.at[pl.ds(sid * 640, 640)],
                        out_lo.at[pl.ds(sid * 640, 640)])
        if with_counts:
            pltpu.sync_copy(cnt_acc.at[pl.ds(sid * 640, 640)],
                            cnt_out.at[pl.ds(sid * 640, 640)])

    @pl.when(cid == 1)
    def _():
        pltpu.sync_copy(acc.at[pl.ds(sid * 640, 640)],
                        out_hi.at[pl.ds(sid * 640, 640)])


_sc_segsum_cnt = pl.kernel(
    functools.partial(_segsum_body, True),
    out_type=[
        jax.ShapeDtypeStruct((NPAD, 128), jnp.float32),
        jax.ShapeDtypeStruct((NPAD, 128), jnp.float32),
        jax.ShapeDtypeStruct((NPAD,), jnp.float32),
    ],
    mesh=_MESH,
    scratch_types=[
        pltpu.VMEM((40, CW), jnp.int32),
        pltpu.VMEM((40, CW), jnp.int32),
        pltpu.VMEM((CW, 128), jnp.float32),
        pltpu.VMEM((CW, 128), jnp.float32),
        pltpu.VMEM((16, 128), jnp.float32),
        pltpu.VMEM((640,), jnp.float32),
        pltpu.VMEM((CW,), jnp.float32),
        pltpu.VMEM_SHARED((NPAD, 128), jnp.float32),
        pltpu.VMEM_SHARED((NPAD,), jnp.float32),
        pltpu.SemaphoreType.DMA,
        pltpu.SemaphoreType.DMA,
    ],
)

_sc_segsum_nocnt = pl.kernel(
    functools.partial(_segsum_body, False),
    out_type=[
        jax.ShapeDtypeStruct((NPAD, 128), jnp.float32),
        jax.ShapeDtypeStruct((NPAD, 128), jnp.float32),
    ],
    mesh=_MESH,
    scratch_types=[
        pltpu.VMEM((40, CW), jnp.int32),
        pltpu.VMEM((40, CW), jnp.int32),
        pltpu.VMEM((CW, 128), jnp.float32),
        pltpu.VMEM((CW, 128), jnp.float32),
        pltpu.VMEM((16, 128), jnp.float32),
        pltpu.VMEM_SHARED((NPAD, 128), jnp.float32),
        pltpu.SemaphoreType.DMA,
        pltpu.SemaphoreType.DMA,
    ],
)


def _gather_model_body(tlo, thi, idx2, out_lo, out_hi, idx_v, rows_v, sem):
    cid = lax.axis_index("c")
    sid = lax.axis_index("s")
    pltpu.sync_copy(idx2.at[pl.ds(sid * 8, 8)], idx_v)

    def run(table, out):
        def step(j, carry):
            pltpu.async_copy(table.at[idx_v.at[j]], rows_v, sem).wait()
            pltpu.sync_copy(rows_v, out.at[pl.ds(sid * 1024 + j * 128, 128)])
            return carry
        lax.fori_loop(0, 8, step, 0)

    @pl.when(cid == 0)
    def _():
        run(tlo, out_lo)

    @pl.when(cid == 1)
    def _():
        run(thi, out_hi)


_sc_gather_model = pl.kernel(
    _gather_model_body,
    out_type=[
        jax.ShapeDtypeStruct((B, 128), jnp.float32),
        jax.ShapeDtypeStruct((B, 128), jnp.float32),
    ],
    mesh=_MESH,
    scratch_types=[
        pltpu.VMEM((8, 128), jnp.int32),
        pltpu.VMEM((128, 128), jnp.float32),
        pltpu.SemaphoreType.DMA,
    ],
)


def _gather_q_body(q_hbm, idx2, out, idx_v, rows_v, sem):
    cid = lax.axis_index("c")
    sid = lax.axis_index("s")
    wid = sid * 2 + cid
    pltpu.sync_copy(idx2.at[pl.ds(wid * 8, 8)], idx_v)

    def step(j, carry):
        pltpu.async_copy(q_hbm.at[idx_v.at[j]], rows_v, sem).wait()
        pltpu.sync_copy(rows_v, out.at[pl.ds(wid * 512 + j * 64, 64)])
        return carry
    lax.fori_loop(0, 8, step, 0)


_sc_gather_q = pl.kernel(
    _gather_q_body,
    out_type=jax.ShapeDtypeStruct((B, TEXT), jnp.float32),
    mesh=_MESH,
    scratch_types=[
        pltpu.VMEM((8, 64), jnp.int32),
        pltpu.VMEM((64, TEXT), jnp.float32),
        pltpu.SemaphoreType.DMA,
    ],
)


def kernel(model_id, prompt_id, edge_index, P, Q, W_proj, b_proj,
           Wl1, Wr1, b1, Wl2, Wr2, b2, W_text, b_text, W_cls, b_cls):
    f32 = jnp.float32
    model_id = model_id.astype(jnp.int32)
    prompt_id = prompt_id.astype(jnp.int32)
    src = edge_index[0].astype(jnp.int32)
    dst = edge_index[1].astype(jnp.int32)
    src2d = jnp.concatenate([src, jnp.zeros((EPAD - E,), jnp.int32)]).reshape(ECH, CW)
    dst2d = jnp.concatenate([dst, jnp.full((EPAD - E,), N, jnp.int32)]).reshape(ECH, CW)
    mid2d = model_id.reshape(128, 128)
    pid2d = prompt_id.reshape(256, 64)

    WprojP = jnp.pad(W_proj, ((0, 0), (0, 24)))
    bprojP = jnp.pad(b_proj, (0, 24)).reshape(1, 256)
    Wl1P = jnp.pad(Wl1, ((0, 24), (0, 0)))
    Wr1P = jnp.pad(Wr1, ((0, 24), (0, 0)))
    b1r = b1.reshape(1, 256)
    Wl2P = jnp.pad(Wl2, ((0, 0), (0, 24)))
    Wr2P = jnp.pad(Wr2, ((0, 0), (0, 24)))
    b2P = jnp.pad(b2, (0, 24)).reshape(1, 256)
    WtP = jnp.pad(W_text, ((0, 0), (0, 24)))
    btP = jnp.pad(b_text, (0, 24)).reshape(1, 256)
    WcP = jnp.pad(W_cls, ((0, 24), (0, 128 - W_cls.shape[1])))
    bcP = jnp.pad(b_cls, (0, 128 - b_cls.shape[0])).reshape(1, 128)

    zf = jnp.zeros((16, 128), f32)
    z1 = jnp.zeros((640,), f32)
    ones = jnp.ones((CW,), f32)

    x_lo, x_hi = _tc_proj(P, WprojP, bprojP)
    s1_lo, s1_hi, cnt = _sc_segsum_cnt(x_lo, x_hi, src2d, dst2d, zf, z1, ones)
    cntc = cnt.reshape(NPAD, 1)
    h1_lo, h1_hi = _tc_conv(s1_lo, s1_hi, x_lo, x_hi, cntc, Wl1P, Wr1P, b1r, True)
    s2_lo, s2_hi = _sc_segsum_nocnt(h1_lo, h1_hi, src2d, dst2d, zf)
    h2_lo, h2_hi = _tc_conv(s2_lo, s2_hi, h1_lo, h1_hi, cntc, Wl2P, Wr2P, b2P, False)
    mr_lo, mr_hi = _sc_gather_model(h2_lo, h2_hi, mid2d)
    qg = _sc_gather_q(Q, pid2d)
    out = _tc_final(qg, mr_lo, mr_hi, WtP, btP, WcP, bcP)
    return out[:, :2]
